# baseline (speedup 1.0000x reference)
import jax
import jax.numpy as jnp
from jax import lax
from jax.experimental import pallas as pl

_TAU = 0.9
_TOP_P = 0.8
_MIN_K = 1
_BB = 128


def _router_body(x_ref, cw_ref, cb_ref, fw_ref, fb_ref, out_ref):
    xb = x_ref[...]
    y = lax.dot_general(xb, cw_ref[...], (((1,), (0,)), ((), ())),
                        preferred_element_type=jnp.float32)
    y = jnp.maximum(y + cb_ref[...][None], 0.0)
    pooled = jnp.mean(y, axis=1)
    logits = (jnp.dot(pooled, fw_ref[...], preferred_element_type=jnp.float32)
              + fb_ref[...])
    s = logits * (1.0 / _TAU)
    s = s - jnp.max(s, axis=-1, keepdims=True)
    e = jnp.exp(s)
    p = e / jnp.sum(e, axis=-1, keepdims=True)
    pi = p[:, :, None]
    pj = p[:, None, :]
    ii = lax.broadcasted_iota(jnp.int32, (_BB, 16, 16), 1)
    jj = lax.broadcasted_iota(jnp.int32, (_BB, 16, 16), 2)
    before = (pj > pi) | ((pj == pi) & (jj <= ii))
    cums = jnp.sum(jnp.where(before, jnp.broadcast_to(pj, before.shape), 0.0),
                   axis=2)
    rank = jnp.sum(before.astype(jnp.int32), axis=2) - 1
    keep = (cums <= _TOP_P) | (rank < _MIN_K)
    masked = jnp.where(keep, p, 0.0)
    denom = jnp.clip(jnp.sum(masked, axis=-1, keepdims=True), 1e-10, None)
    out_ref[...] = masked / denom


def kernel(patch, conv_w, conv_b, fc_w, fc_b, layer_idx, threshold):
    B, C, H, W = patch.shape
    x = patch.reshape(B, C, H * W)
    return pl.pallas_call(
        _router_body,
        grid=(B // _BB,),
        in_specs=[
            pl.BlockSpec((_BB, C, H * W), lambda i: (i, 0, 0)),
            pl.BlockSpec((C, 128), lambda i: (0, 0)),
            pl.BlockSpec((1, 128), lambda i: (0, 0)),
            pl.BlockSpec((128, 16), lambda i: (0, 0)),
            pl.BlockSpec((1, 16), lambda i: (0, 0)),
        ],
        out_specs=pl.BlockSpec((_BB, 16), lambda i: (i, 0)),
        out_shape=jax.ShapeDtypeStruct((B, 16), jnp.float32),
    )(x, conv_w.T, conv_b.reshape(1, 128), fc_w.T, fc_b.reshape(1, 16))


# speedup vs baseline: 1.0008x; 1.0008x over previous
import jax
import jax.numpy as jnp
from jax import lax
from jax.experimental import pallas as pl

_TAU = 0.9
_TOP_P = 0.8
_MIN_K = 1
_BB = 128


def _router_body(x_ref, cwa_ref, cwb_ref, cb_ref, fw_ref, fb_ref, out_ref):
    xb = x_ref[...]
    dn = (((1,), (0,)), ((), ()))
    y = (lax.dot_general(xb[:, :, 0:64], cwa_ref[...], dn,
                         preferred_element_type=jnp.float32)
         + lax.dot_general(xb[:, :, 64:128], cwb_ref[...], dn,
                           preferred_element_type=jnp.float32))
    y = jnp.maximum(y + cb_ref[...][None], 0.0)
    pooled = jnp.mean(y, axis=1)
    logits = (jnp.dot(pooled, fw_ref[...], preferred_element_type=jnp.float32)
              + fb_ref[...])
    s = logits * (1.0 / _TAU)
    s = s - jnp.max(s, axis=-1, keepdims=True)
    e = jnp.exp(s)
    p = e / jnp.sum(e, axis=-1, keepdims=True)
    pi = p[:, :, None]
    pj = p[:, None, :]
    ii = lax.broadcasted_iota(jnp.int32, (_BB, 16, 16), 1)
    jj = lax.broadcasted_iota(jnp.int32, (_BB, 16, 16), 2)
    before = (pj > pi) | ((pj == pi) & (jj <= ii))
    cums = jnp.sum(jnp.where(before, jnp.broadcast_to(pj, before.shape), 0.0),
                   axis=2)
    rank = jnp.sum(before.astype(jnp.int32), axis=2) - 1
    keep = (cums <= _TOP_P) | (rank < _MIN_K)
    masked = jnp.where(keep, p, 0.0)
    denom = jnp.clip(jnp.sum(masked, axis=-1, keepdims=True), 1e-10, None)
    out_ref[...] = masked / denom


def kernel(patch, conv_w, conv_b, fc_w, fc_b, layer_idx, threshold):
    B, C, H, W = patch.shape
    x = patch.reshape(B, (C * H * W) // 128, 128)
    return pl.pallas_call(
        _router_body,
        grid=(B // _BB,),
        in_specs=[
            pl.BlockSpec((_BB, (C * H * W) // 128, 128), lambda i: (i, 0, 0)),
            pl.BlockSpec((C // 2, 128), lambda i: (0, 0)),
            pl.BlockSpec((C // 2, 128), lambda i: (0, 0)),
            pl.BlockSpec((1, 128), lambda i: (0, 0)),
            pl.BlockSpec((128, 16), lambda i: (0, 0)),
            pl.BlockSpec((1, 16), lambda i: (0, 0)),
        ],
        out_specs=pl.BlockSpec((_BB, 16), lambda i: (i, 0)),
        out_shape=jax.ShapeDtypeStruct((B, 16), jnp.float32),
    )(x, conv_w.T[0::2, :], conv_w.T[1::2, :], conv_b.reshape(1, 128),
      fc_w.T, fc_b.reshape(1, 16))
